# baseline (speedup 1.0000x reference)
import functools

import jax
import jax.numpy as jnp
from jax import lax
from jax.experimental import pallas as pl
from jax.experimental.pallas import tpu as pltpu
from jax.experimental.pallas import tpu_sc as plsc

N = 10000
E = 320000
D = 128
NC, NS = 2, 16
NB, K = 80, 125
NBC = 10
STRIPE = N // NS
CW = 16
ZR = 25

f32 = jnp.float32
bf16 = jnp.bfloat16

_MESH = plsc.VectorSubcoreMesh(core_axis_name="c", subcore_axis_name="s")
_SC_PARAMS = pltpu.CompilerParams(use_tc_tiling_on_sc=False)


def _zero_vmem_2d(ref, rows, cols):
    step = 32 if ref.dtype == bf16 else 16
    @pl.loop(0, rows)
    def _(r):
        @pl.loop(0, cols, step=step)
        def _(col):
            ref[r, pl.ds(col, step)] = jnp.zeros((step,), ref.dtype)


@functools.partial(
    pl.kernel,
    out_type=(
        jax.ShapeDtypeStruct((NC, N, D), bf16),
        jax.ShapeDtypeStruct((NC, N, CW), f32),
    ),
    mesh=_MESH,
    scratch_types=[
        pltpu.VMEM_SHARED((N, D), bf16),
        pltpu.VMEM_SHARED((N, CW), f32),
        pltpu.VMEM((NBC, K), jnp.int32),
        pltpu.VMEM((NBC, K), jnp.int32),
        pltpu.VMEM((K, D), bf16),
        pltpu.VMEM((K, D), bf16),
        pltpu.VMEM((ZR, D), bf16),
        pltpu.VMEM((ZR, CW), f32),
        pltpu.VMEM((K, CW), f32),
        pltpu.SemaphoreType.DMA,
        pltpu.SemaphoreType.DMA,
    ],
    compiler_params=_SC_PARAMS,
)
def _sc_agg_counts(x_hbm, src_hbm, dst_hbm, out_hbm, cnt_hbm,
                   acc, cacc, srcv, dstv, rows_a, rows_b, zbuf, zcnt, ones,
                   sem_a, sem_b):
    c = lax.axis_index("c")
    s = lax.axis_index("s")
    _zero_vmem_2d(zbuf, ZR, D)
    _zero_vmem_2d(zcnt, ZR, CW)

    @pl.loop(0, K)
    def _(r):
        ones[r, :] = jnp.ones((CW,), f32)

    row0 = s * STRIPE
    @pl.loop(0, STRIPE // ZR)
    def _(i):
        pltpu.sync_copy(zbuf, acc.at[pl.ds(row0 + i * ZR, ZR)])
        pltpu.sync_copy(zcnt, cacc.at[pl.ds(row0 + i * ZR, ZR)])
    plsc.subcore_barrier()

    bufs = (rows_a, rows_b)
    sems = (sem_a, sem_b)

    @pl.loop(0, NB // NBC)
    def _(g):
        pltpu.sync_copy(src_hbm.at[c, s, pl.ds(g * NBC, NBC)], srcv)
        pltpu.sync_copy(dst_hbm.at[c, s, pl.ds(g * NBC, NBC)], dstv)
        pltpu.async_copy(x_hbm.at[srcv.at[0]], bufs[0], sems[0])
        for b in range(NBC):
            if b + 1 < NBC:
                pltpu.async_copy(x_hbm.at[srcv.at[b + 1]],
                                 bufs[(b + 1) % 2], sems[(b + 1) % 2])
            pltpu.make_async_copy(x_hbm.at[srcv.at[b]],
                                  bufs[b % 2], sems[b % 2]).wait()
            pltpu.sync_copy(bufs[b % 2], acc.at[dstv.at[b]], add=True)
            pltpu.sync_copy(ones, cacc.at[dstv.at[b]], add=True)

    plsc.subcore_barrier()
    pltpu.sync_copy(acc.at[pl.ds(row0, STRIPE)],
                    out_hbm.at[c, pl.ds(row0, STRIPE)])
    pltpu.sync_copy(cacc.at[pl.ds(row0, STRIPE)],
                    cnt_hbm.at[c, pl.ds(row0, STRIPE)])


@functools.partial(
    pl.kernel,
    out_type=jax.ShapeDtypeStruct((NC, N, D), bf16),
    mesh=_MESH,
    scratch_types=[
        pltpu.VMEM_SHARED((N, D), bf16),
        pltpu.VMEM((NBC, K), jnp.int32),
        pltpu.VMEM((NBC, K), jnp.int32),
        pltpu.VMEM((K, D), bf16),
        pltpu.VMEM((K, D), bf16),
        pltpu.VMEM((ZR, D), bf16),
        pltpu.SemaphoreType.DMA,
        pltpu.SemaphoreType.DMA,
    ],
    compiler_params=_SC_PARAMS,
)
def _sc_agg(x_hbm, src_hbm, dst_hbm, out_hbm,
            acc, srcv, dstv, rows_a, rows_b, zbuf, sem_a, sem_b):
    c = lax.axis_index("c")
    s = lax.axis_index("s")
    _zero_vmem_2d(zbuf, ZR, D)

    row0 = s * STRIPE
    @pl.loop(0, STRIPE // ZR)
    def _(i):
        pltpu.sync_copy(zbuf, acc.at[pl.ds(row0 + i * ZR, ZR)])
    plsc.subcore_barrier()

    bufs = (rows_a, rows_b)
    sems = (sem_a, sem_b)

    @pl.loop(0, NB // NBC)
    def _(g):
        pltpu.sync_copy(src_hbm.at[c, s, pl.ds(g * NBC, NBC)], srcv)
        pltpu.sync_copy(dst_hbm.at[c, s, pl.ds(g * NBC, NBC)], dstv)
        pltpu.async_copy(x_hbm.at[srcv.at[0]], bufs[0], sems[0])
        for b in range(NBC):
            if b + 1 < NBC:
                pltpu.async_copy(x_hbm.at[srcv.at[b + 1]],
                                 bufs[(b + 1) % 2], sems[(b + 1) % 2])
            pltpu.make_async_copy(x_hbm.at[srcv.at[b]],
                                  bufs[b % 2], sems[b % 2]).wait()
            pltpu.sync_copy(bufs[b % 2], acc.at[dstv.at[b]], add=True)

    plsc.subcore_barrier()
    pltpu.sync_copy(acc.at[pl.ds(row0, STRIPE)],
                    out_hbm.at[c, pl.ds(row0, STRIPE)])


BN = 1000


_DNUMS = (((1,), (1,)), ((), ()))


def _tc_self_body(x_ref, wr_ref, br_ref, o_ref):
    o_ref[...] = (lax.dot_general(x_ref[...].astype(bf16),
                                  wr_ref[...].astype(bf16), _DNUMS,
                                  preferred_element_type=f32)
                  + br_ref[...])


def _tc_self(x, Wr, br):
    return pl.pallas_call(
        _tc_self_body,
        grid=(N // BN,),
        in_specs=[
            pl.BlockSpec((BN, D), lambda i: (i, 0)),
            pl.BlockSpec((D, D), lambda i: (0, 0)),
            pl.BlockSpec((1, D), lambda i: (0, 0)),
        ],
        out_specs=pl.BlockSpec((BN, D), lambda i: (i, 0)),
        out_shape=jax.ShapeDtypeStruct((N, D), f32),
    )(x, Wr, br)


def _tc_layer_body(act, p_ref, cnt_ref, st_ref, wl_ref, *o_refs):
    ssum = p_ref[0].astype(f32) + p_ref[1].astype(f32)
    count = cnt_ref[0, :, 0:1] + cnt_ref[1, :, 0:1]
    mean = (ssum / jnp.maximum(count, 1.0)).astype(bf16)
    y = (lax.dot_general(mean, wl_ref[...].astype(bf16), _DNUMS,
                         preferred_element_type=f32)
         + st_ref[...])
    if act == "relu":
        h = jnp.maximum(y, 0.0)
        o_refs[0][...] = h
        o_refs[1][...] = h.astype(bf16)
    else:
        m = jnp.max(y, axis=1, keepdims=True)
        lse = m + jnp.log(jnp.sum(jnp.exp(y - m), axis=1, keepdims=True))
        o_refs[0][...] = y - lse


def _tc_layer(p, cnt, st, Wl, act):
    if act == "relu":
        out_specs = [pl.BlockSpec((BN, D), lambda i: (i, 0)),
                     pl.BlockSpec((BN, D), lambda i: (i, 0))]
        out_shape = [jax.ShapeDtypeStruct((N, D), f32),
                     jax.ShapeDtypeStruct((N, D), bf16)]
    else:
        out_specs = pl.BlockSpec((BN, D), lambda i: (i, 0))
        out_shape = jax.ShapeDtypeStruct((N, D), f32)
    return pl.pallas_call(
        functools.partial(_tc_layer_body, act),
        grid=(N // BN,),
        in_specs=[
            pl.BlockSpec((NC, BN, D), lambda i: (0, i, 0)),
            pl.BlockSpec((NC, BN, CW), lambda i: (0, i, 0)),
            pl.BlockSpec((BN, D), lambda i: (i, 0)),
            pl.BlockSpec((D, D), lambda i: (0, 0)),
        ],
        out_specs=out_specs,
        out_shape=out_shape,
    )(p, cnt, st, Wl)


def kernel(x, edge_index, W1l, b1, W1r, W2l, b2, W2r):
    ei = edge_index.astype(jnp.int32)
    src = ei[0].reshape(NC, NS, NB, K)
    dst = ei[1].reshape(NC, NS, NB, K)

    p1, cnt = _sc_agg_counts(x.astype(bf16), src, dst)
    st1 = _tc_self(x, W1r, b1.reshape(1, D))
    h, hb = _tc_layer(p1, cnt, st1, W1l, "relu")
    p2 = _sc_agg(hb, src, dst)
    st2 = _tc_self(h, W2r, b2.reshape(1, D))
    return _tc_layer(p2, cnt, st2, W2l, "logsoftmax")


# speedup vs baseline: 1.0186x; 1.0186x over previous
import functools

import jax
import jax.numpy as jnp
from jax import lax
from jax.experimental import pallas as pl
from jax.experimental.pallas import tpu as pltpu
from jax.experimental.pallas import tpu_sc as plsc

N = 10000
E = 320000
D = 128
NC, NS = 2, 16
NB, K = 40, 250
NBC = 5
STRIPE = N // NS
CW = 16
ZR = 25

f32 = jnp.float32
bf16 = jnp.bfloat16

_MESH = plsc.VectorSubcoreMesh(core_axis_name="c", subcore_axis_name="s")
_SC_PARAMS = pltpu.CompilerParams(use_tc_tiling_on_sc=False)


def _zero_vmem_2d(ref, rows, cols):
    step = 32 if ref.dtype == bf16 else 16
    @pl.loop(0, rows)
    def _(r):
        @pl.loop(0, cols, step=step)
        def _(col):
            ref[r, pl.ds(col, step)] = jnp.zeros((step,), ref.dtype)


@functools.partial(
    pl.kernel,
    out_type=(
        jax.ShapeDtypeStruct((NC, N, D), bf16),
        jax.ShapeDtypeStruct((NC, N, CW), f32),
    ),
    mesh=_MESH,
    scratch_types=[
        pltpu.VMEM_SHARED((N, D), bf16),
        pltpu.VMEM_SHARED((N, CW), f32),
        pltpu.VMEM((NBC, K), jnp.int32),
        pltpu.VMEM((NBC, K), jnp.int32),
        pltpu.VMEM((K, D), bf16),
        pltpu.VMEM((K, D), bf16),
        pltpu.VMEM((ZR, D), bf16),
        pltpu.VMEM((ZR, CW), f32),
        pltpu.VMEM((K, CW), f32),
        pltpu.SemaphoreType.DMA,
        pltpu.SemaphoreType.DMA,
    ],
    compiler_params=_SC_PARAMS,
)
def _sc_agg_counts(x_hbm, src_hbm, dst_hbm, out_hbm, cnt_hbm,
                   acc, cacc, srcv, dstv, rows_a, rows_b, zbuf, zcnt, ones,
                   sem_a, sem_b):
    c = lax.axis_index("c")
    s = lax.axis_index("s")
    _zero_vmem_2d(zbuf, ZR, D)
    _zero_vmem_2d(zcnt, ZR, CW)

    @pl.loop(0, K)
    def _(r):
        ones[r, :] = jnp.ones((CW,), f32)

    row0 = s * STRIPE
    @pl.loop(0, STRIPE // ZR)
    def _(i):
        pltpu.sync_copy(zbuf, acc.at[pl.ds(row0 + i * ZR, ZR)])
        pltpu.sync_copy(zcnt, cacc.at[pl.ds(row0 + i * ZR, ZR)])
    plsc.subcore_barrier()

    bufs = (rows_a, rows_b)
    sems = (sem_a, sem_b)

    @pl.loop(0, NB // NBC)
    def _(g):
        pltpu.sync_copy(src_hbm.at[c, s, pl.ds(g * NBC, NBC)], srcv)
        pltpu.sync_copy(dst_hbm.at[c, s, pl.ds(g * NBC, NBC)], dstv)
        pltpu.async_copy(x_hbm.at[srcv.at[0]], bufs[0], sems[0])
        for b in range(NBC):
            if b + 1 < NBC:
                pltpu.async_copy(x_hbm.at[srcv.at[b + 1]],
                                 bufs[(b + 1) % 2], sems[(b + 1) % 2])
            pltpu.make_async_copy(x_hbm.at[srcv.at[b]],
                                  bufs[b % 2], sems[b % 2]).wait()
            pltpu.sync_copy(bufs[b % 2], acc.at[dstv.at[b]], add=True)
            pltpu.sync_copy(ones, cacc.at[dstv.at[b]], add=True)

    plsc.subcore_barrier()
    pltpu.sync_copy(acc.at[pl.ds(row0, STRIPE)],
                    out_hbm.at[c, pl.ds(row0, STRIPE)])
    pltpu.sync_copy(cacc.at[pl.ds(row0, STRIPE)],
                    cnt_hbm.at[c, pl.ds(row0, STRIPE)])


@functools.partial(
    pl.kernel,
    out_type=jax.ShapeDtypeStruct((NC, N, D), bf16),
    mesh=_MESH,
    scratch_types=[
        pltpu.VMEM_SHARED((N, D), bf16),
        pltpu.VMEM((NBC, K), jnp.int32),
        pltpu.VMEM((NBC, K), jnp.int32),
        pltpu.VMEM((K, D), bf16),
        pltpu.VMEM((K, D), bf16),
        pltpu.VMEM((ZR, D), bf16),
        pltpu.SemaphoreType.DMA,
        pltpu.SemaphoreType.DMA,
    ],
    compiler_params=_SC_PARAMS,
)
def _sc_agg(x_hbm, src_hbm, dst_hbm, out_hbm,
            acc, srcv, dstv, rows_a, rows_b, zbuf, sem_a, sem_b):
    c = lax.axis_index("c")
    s = lax.axis_index("s")
    _zero_vmem_2d(zbuf, ZR, D)

    row0 = s * STRIPE
    @pl.loop(0, STRIPE // ZR)
    def _(i):
        pltpu.sync_copy(zbuf, acc.at[pl.ds(row0 + i * ZR, ZR)])
    plsc.subcore_barrier()

    bufs = (rows_a, rows_b)
    sems = (sem_a, sem_b)

    @pl.loop(0, NB // NBC)
    def _(g):
        pltpu.sync_copy(src_hbm.at[c, s, pl.ds(g * NBC, NBC)], srcv)
        pltpu.sync_copy(dst_hbm.at[c, s, pl.ds(g * NBC, NBC)], dstv)
        pltpu.async_copy(x_hbm.at[srcv.at[0]], bufs[0], sems[0])
        for b in range(NBC):
            if b + 1 < NBC:
                pltpu.async_copy(x_hbm.at[srcv.at[b + 1]],
                                 bufs[(b + 1) % 2], sems[(b + 1) % 2])
            pltpu.make_async_copy(x_hbm.at[srcv.at[b]],
                                  bufs[b % 2], sems[b % 2]).wait()
            pltpu.sync_copy(bufs[b % 2], acc.at[dstv.at[b]], add=True)

    plsc.subcore_barrier()
    pltpu.sync_copy(acc.at[pl.ds(row0, STRIPE)],
                    out_hbm.at[c, pl.ds(row0, STRIPE)])


BN = 1000


_DNUMS = (((1,), (1,)), ((), ()))


def _tc_self_body(x_ref, wr_ref, br_ref, o_ref):
    o_ref[...] = (lax.dot_general(x_ref[...].astype(bf16),
                                  wr_ref[...].astype(bf16), _DNUMS,
                                  preferred_element_type=f32)
                  + br_ref[...])


def _tc_self(x, Wr, br):
    return pl.pallas_call(
        _tc_self_body,
        grid=(N // BN,),
        in_specs=[
            pl.BlockSpec((BN, D), lambda i: (i, 0)),
            pl.BlockSpec((D, D), lambda i: (0, 0)),
            pl.BlockSpec((1, D), lambda i: (0, 0)),
        ],
        out_specs=pl.BlockSpec((BN, D), lambda i: (i, 0)),
        out_shape=jax.ShapeDtypeStruct((N, D), f32),
    )(x, Wr, br)


def _tc_layer_body(act, p_ref, cnt_ref, st_ref, wl_ref, *o_refs):
    ssum = p_ref[0].astype(f32) + p_ref[1].astype(f32)
    count = cnt_ref[0, :, 0:1] + cnt_ref[1, :, 0:1]
    mean = (ssum / jnp.maximum(count, 1.0)).astype(bf16)
    y = (lax.dot_general(mean, wl_ref[...].astype(bf16), _DNUMS,
                         preferred_element_type=f32)
         + st_ref[...])
    if act == "relu":
        h = jnp.maximum(y, 0.0)
        o_refs[0][...] = h
        o_refs[1][...] = h.astype(bf16)
    else:
        m = jnp.max(y, axis=1, keepdims=True)
        lse = m + jnp.log(jnp.sum(jnp.exp(y - m), axis=1, keepdims=True))
        o_refs[0][...] = y - lse


def _tc_layer(p, cnt, st, Wl, act):
    if act == "relu":
        out_specs = [pl.BlockSpec((BN, D), lambda i: (i, 0)),
                     pl.BlockSpec((BN, D), lambda i: (i, 0))]
        out_shape = [jax.ShapeDtypeStruct((N, D), f32),
                     jax.ShapeDtypeStruct((N, D), bf16)]
    else:
        out_specs = pl.BlockSpec((BN, D), lambda i: (i, 0))
        out_shape = jax.ShapeDtypeStruct((N, D), f32)
    return pl.pallas_call(
        functools.partial(_tc_layer_body, act),
        grid=(N // BN,),
        in_specs=[
            pl.BlockSpec((NC, BN, D), lambda i: (0, i, 0)),
            pl.BlockSpec((NC, BN, CW), lambda i: (0, i, 0)),
            pl.BlockSpec((BN, D), lambda i: (i, 0)),
            pl.BlockSpec((D, D), lambda i: (0, 0)),
        ],
        out_specs=out_specs,
        out_shape=out_shape,
    )(p, cnt, st, Wl)


def kernel(x, edge_index, W1l, b1, W1r, W2l, b2, W2r):
    ei = edge_index.astype(jnp.int32)
    src = ei[0].reshape(NC, NS, NB, K)
    dst = ei[1].reshape(NC, NS, NB, K)

    p1, cnt = _sc_agg_counts(x.astype(bf16), src, dst)
    st1 = _tc_self(x, W1r, b1.reshape(1, D))
    h, hb = _tc_layer(p1, cnt, st1, W1l, "relu")
    p2 = _sc_agg(hb, src, dst)
    st2 = _tc_self(h, W2r, b2.reshape(1, D))
    return _tc_layer(p2, cnt, st2, W2l, "logsoftmax")


# speedup vs baseline: 1.0320x; 1.0132x over previous
import functools

import jax
import jax.numpy as jnp
from jax import lax
from jax.experimental import pallas as pl
from jax.experimental.pallas import tpu as pltpu
from jax.experimental.pallas import tpu_sc as plsc

N = 10000
E = 320000
D = 128
NC, NS = 2, 16
NB, K = 40, 250
NBC = 5
STRIPE = N // NS
CW = 16
ZR = 25

f32 = jnp.float32
bf16 = jnp.bfloat16

_MESH = plsc.VectorSubcoreMesh(core_axis_name="c", subcore_axis_name="s")
_SC_PARAMS = pltpu.CompilerParams(use_tc_tiling_on_sc=False)


def _zero_vmem_2d(ref, rows, cols):
    step = 32 if ref.dtype == bf16 else 16
    @pl.loop(0, rows)
    def _(r):
        @pl.loop(0, cols, step=step)
        def _(col):
            ref[r, pl.ds(col, step)] = jnp.zeros((step,), ref.dtype)


@functools.partial(
    pl.kernel,
    out_type=(
        jax.ShapeDtypeStruct((NC, N, D), bf16),
        jax.ShapeDtypeStruct((NC, N, CW), f32),
    ),
    mesh=_MESH,
    scratch_types=[
        pltpu.VMEM_SHARED((N, D), bf16),
        pltpu.VMEM_SHARED((N, CW), f32),
        pltpu.VMEM((NBC, K), jnp.int32),
        pltpu.VMEM((NBC, K), jnp.int32),
        pltpu.VMEM((K, D), bf16),
        pltpu.VMEM((K, D), bf16),
        pltpu.VMEM((ZR, D), bf16),
        pltpu.VMEM((ZR, CW), f32),
        pltpu.VMEM((K, CW), f32),
        pltpu.SemaphoreType.DMA,
        pltpu.SemaphoreType.DMA,
    ],
    compiler_params=_SC_PARAMS,
)
def _sc_agg_counts(x_hbm, src_hbm, dst_hbm, out_hbm, cnt_hbm,
                   acc, cacc, srcv, dstv, rows_a, rows_b, zbuf, zcnt, ones,
                   sem_a, sem_b):
    c = lax.axis_index("c")
    s = lax.axis_index("s")
    _zero_vmem_2d(zbuf, ZR, D)
    _zero_vmem_2d(zcnt, ZR, CW)

    @pl.loop(0, K)
    def _(r):
        ones[r, :] = jnp.ones((CW,), f32)

    row0 = s * STRIPE
    @pl.loop(0, STRIPE // ZR)
    def _(i):
        pltpu.sync_copy(zbuf, acc.at[pl.ds(row0 + i * ZR, ZR)])
        pltpu.sync_copy(zcnt, cacc.at[pl.ds(row0 + i * ZR, ZR)])
    plsc.subcore_barrier()

    bufs = (rows_a, rows_b)
    sems = (sem_a, sem_b)

    @pl.loop(0, NB // NBC)
    def _(g):
        pltpu.sync_copy(src_hbm.at[c, s, pl.ds(g * NBC, NBC)], srcv)
        pltpu.sync_copy(dst_hbm.at[c, s, pl.ds(g * NBC, NBC)], dstv)
        pltpu.async_copy(x_hbm.at[srcv.at[0]], bufs[0], sems[0])
        for b in range(NBC):
            if b + 1 < NBC:
                pltpu.async_copy(x_hbm.at[srcv.at[b + 1]],
                                 bufs[(b + 1) % 2], sems[(b + 1) % 2])
            pltpu.make_async_copy(x_hbm.at[srcv.at[b]],
                                  bufs[b % 2], sems[b % 2]).wait()
            pltpu.sync_copy(bufs[b % 2], acc.at[dstv.at[b]], add=True)
            pltpu.sync_copy(ones, cacc.at[dstv.at[b]], add=True)

    plsc.subcore_barrier()
    pltpu.sync_copy(acc.at[pl.ds(row0, STRIPE)],
                    out_hbm.at[c, pl.ds(row0, STRIPE)])
    pltpu.sync_copy(cacc.at[pl.ds(row0, STRIPE)],
                    cnt_hbm.at[c, pl.ds(row0, STRIPE)])


@functools.partial(
    pl.kernel,
    out_type=jax.ShapeDtypeStruct((NC, N, D), bf16),
    mesh=_MESH,
    scratch_types=[
        pltpu.VMEM_SHARED((N, D), bf16),
        pltpu.VMEM((NBC, K), jnp.int32),
        pltpu.VMEM((NBC, K), jnp.int32),
        pltpu.VMEM((K, D), bf16),
        pltpu.VMEM((K, D), bf16),
        pltpu.VMEM((ZR, D), bf16),
        pltpu.SemaphoreType.DMA,
        pltpu.SemaphoreType.DMA,
    ],
    compiler_params=_SC_PARAMS,
)
def _sc_agg(x_hbm, src_hbm, dst_hbm, out_hbm,
            acc, srcv, dstv, rows_a, rows_b, zbuf, sem_a, sem_b):
    c = lax.axis_index("c")
    s = lax.axis_index("s")
    _zero_vmem_2d(zbuf, ZR, D)

    row0 = s * STRIPE
    @pl.loop(0, STRIPE // ZR)
    def _(i):
        pltpu.sync_copy(zbuf, acc.at[pl.ds(row0 + i * ZR, ZR)])
    plsc.subcore_barrier()

    bufs = (rows_a, rows_b)
    sems = (sem_a, sem_b)

    @pl.loop(0, NB // NBC)
    def _(g):
        pltpu.sync_copy(src_hbm.at[c, s, pl.ds(g * NBC, NBC)], srcv)
        pltpu.sync_copy(dst_hbm.at[c, s, pl.ds(g * NBC, NBC)], dstv)
        pltpu.async_copy(x_hbm.at[srcv.at[0]], bufs[0], sems[0])
        for b in range(NBC):
            if b + 1 < NBC:
                pltpu.async_copy(x_hbm.at[srcv.at[b + 1]],
                                 bufs[(b + 1) % 2], sems[(b + 1) % 2])
            pltpu.make_async_copy(x_hbm.at[srcv.at[b]],
                                  bufs[b % 2], sems[b % 2]).wait()
            pltpu.sync_copy(bufs[b % 2], acc.at[dstv.at[b]], add=True)

    plsc.subcore_barrier()
    pltpu.sync_copy(acc.at[pl.ds(row0, STRIPE)],
                    out_hbm.at[c, pl.ds(row0, STRIPE)])


BN = 2000


_DNUMS = (((1,), (1,)), ((), ()))


def _tc_self_body(x_ref, wr_ref, br_ref, o_ref):
    o_ref[...] = (lax.dot_general(x_ref[...].astype(bf16),
                                  wr_ref[...].astype(bf16), _DNUMS,
                                  preferred_element_type=f32)
                  + br_ref[...])


def _tc_self(x, Wr, br):
    return pl.pallas_call(
        _tc_self_body,
        grid=(N // BN,),
        in_specs=[
            pl.BlockSpec((BN, D), lambda i: (i, 0)),
            pl.BlockSpec((D, D), lambda i: (0, 0)),
            pl.BlockSpec((1, D), lambda i: (0, 0)),
        ],
        out_specs=pl.BlockSpec((BN, D), lambda i: (i, 0)),
        out_shape=jax.ShapeDtypeStruct((N, D), f32),
    )(x, Wr, br)


def _tc_layer_body(act, p_ref, cnt_ref, st_ref, wl_ref, *o_refs):
    ssum = p_ref[0].astype(f32) + p_ref[1].astype(f32)
    count = cnt_ref[0, :, 0:1] + cnt_ref[1, :, 0:1]
    mean = (ssum / jnp.maximum(count, 1.0)).astype(bf16)
    y = (lax.dot_general(mean, wl_ref[...].astype(bf16), _DNUMS,
                         preferred_element_type=f32)
         + st_ref[...])
    if act == "relu":
        h = jnp.maximum(y, 0.0)
        o_refs[0][...] = h
        o_refs[1][...] = h.astype(bf16)
    else:
        m = jnp.max(y, axis=1, keepdims=True)
        lse = m + jnp.log(jnp.sum(jnp.exp(y - m), axis=1, keepdims=True))
        o_refs[0][...] = y - lse


def _tc_layer(p, cnt, st, Wl, act):
    if act == "relu":
        out_specs = [pl.BlockSpec((BN, D), lambda i: (i, 0)),
                     pl.BlockSpec((BN, D), lambda i: (i, 0))]
        out_shape = [jax.ShapeDtypeStruct((N, D), f32),
                     jax.ShapeDtypeStruct((N, D), bf16)]
    else:
        out_specs = pl.BlockSpec((BN, D), lambda i: (i, 0))
        out_shape = jax.ShapeDtypeStruct((N, D), f32)
    return pl.pallas_call(
        functools.partial(_tc_layer_body, act),
        grid=(N // BN,),
        in_specs=[
            pl.BlockSpec((NC, BN, D), lambda i: (0, i, 0)),
            pl.BlockSpec((NC, BN, CW), lambda i: (0, i, 0)),
            pl.BlockSpec((BN, D), lambda i: (i, 0)),
            pl.BlockSpec((D, D), lambda i: (0, 0)),
        ],
        out_specs=out_specs,
        out_shape=out_shape,
    )(p, cnt, st, Wl)


def kernel(x, edge_index, W1l, b1, W1r, W2l, b2, W2r):
    ei = edge_index.astype(jnp.int32)
    src = ei[0].reshape(NC, NS, NB, K)
    dst = ei[1].reshape(NC, NS, NB, K)

    p1, cnt = _sc_agg_counts(x.astype(bf16), src, dst)
    st1 = _tc_self(x, W1r, b1.reshape(1, D))
    h, hb = _tc_layer(p1, cnt, st1, W1l, "relu")
    p2 = _sc_agg(hb, src, dst)
    st2 = _tc_self(h, W2r, b2.reshape(1, D))
    return _tc_layer(p2, cnt, st2, W2l, "logsoftmax")
